# Initial kernel scaffold; baseline (speedup 1.0000x reference)
import jax
import jax.numpy as jnp
from jax.experimental import pallas as pl


def kernel(x, edge_index, edge_attr, W_enc, b_enc, W_c0, b_c0, W_c1, b_c1, W_c2, b_c2, W_p0, b_p0, W_p1, b_p1, W_p2, b_p2):
    raise NotImplementedError("write your pallas kernel here")


# speedup vs baseline: 4.6526x; 4.6526x over previous
import functools

import jax
import jax.numpy as jnp
from jax import lax
from jax.experimental import pallas as pl
from jax.experimental.pallas import tpu as pltpu
from jax.experimental.pallas import tpu_sc as plsc

N = 10000
E = 320000
F_IN = 128
F_E = 16
H = 64
C = 10

NC = 2
NS = 16
NW = NC * NS

NP = 10240
EPAD = 327680
EPW = EPAD // NW
CHUNK = 128
NCHUNK = EPW // CHUNK
RS = NP // NS

BN = 1024
BE = 2048

_mesh = plsc.VectorSubcoreMesh(
    core_axis_name="c", subcore_axis_name="s", num_cores=NC, num_subcores=NS
)


def _zero2d(ref, nrows, ncols):
    z16 = jnp.zeros((16,), jnp.float32)

    def body(i, carry):
        r = i // (ncols // 16)
        j = (i % (ncols // 16)) * 16
        ref[r, pl.ds(j, 16)] = z16
        return carry

    lax.fori_loop(0, nrows * (ncols // 16), body, 0)


@functools.partial(
    pl.kernel,
    out_type=jax.ShapeDtypeStruct((NW, NP), jnp.float32),
    mesh=_mesh,
    compiler_params=pltpu.CompilerParams(needs_layout_passes=False, use_tc_tiling_on_sc=False),
    scratch_types=[
        pltpu.VMEM((NP,), jnp.float32),
        pltpu.VMEM((EPW,), jnp.int32),
    ],
)
def _sc_degree(dst_hbm, out_hbm, degv, didx):
    c = lax.axis_index("c")
    s = lax.axis_index("s")
    w = c * NS + s
    z16 = jnp.zeros((16,), jnp.float32)

    def zb(i, carry):
        degv[pl.ds(i * 16, 16)] = z16
        return carry

    lax.fori_loop(0, NP // 16, zb, 0)

    pltpu.sync_copy(dst_hbm.at[pl.ds(w * EPW, EPW)], didx)
    ones = jnp.ones((16,), jnp.float32)

    def body(i, carry):
        idx = didx[pl.ds(i * 16, 16)]
        plsc.addupdate_scatter(degv, [idx], ones)
        return carry

    lax.fori_loop(0, EPW // 16, body, 0)
    pltpu.sync_copy(degv, out_hbm.at[w])


@functools.partial(
    pl.kernel,
    out_type=jax.ShapeDtypeStruct((NC, NP, H), jnp.float32),
    mesh=_mesh,
    compiler_params=pltpu.CompilerParams(needs_layout_passes=False, use_tc_tiling_on_sc=False),
    scratch_types=[
        pltpu.VMEM((CHUNK,), jnp.int32),
        pltpu.VMEM((CHUNK,), jnp.int32),
        pltpu.VMEM((CHUNK, H), jnp.float32),
        pltpu.VMEM((CHUNK, H), jnp.float32),
        pltpu.VMEM_SHARED((NP, H), jnp.float32),
    ],
)
def _sc_conv(g_hbm, src_hbm, dst_hbm, out_hbm, sidx, didx, rows, zbuf, acc):
    c = lax.axis_index("c")
    s = lax.axis_index("s")
    w = c * NS + s
    _zero2d(zbuf, CHUNK, H)
    base_r = s * RS
    for k in range(RS // CHUNK):
        pltpu.sync_copy(zbuf, acc.at[pl.ds(base_r + k * CHUNK, CHUNK)])
    plsc.subcore_barrier()

    base_e = w * EPW

    def body(i, carry):
        off = base_e + i * CHUNK
        pltpu.sync_copy(src_hbm.at[pl.ds(off, CHUNK)], sidx)
        pltpu.sync_copy(dst_hbm.at[pl.ds(off, CHUNK)], didx)
        pltpu.sync_copy(g_hbm.at[sidx], rows)
        pltpu.sync_copy(rows, acc.at[didx], add=True)
        return carry

    lax.fori_loop(0, NCHUNK, body, 0)
    plsc.subcore_barrier()
    pltpu.sync_copy(acc.at[pl.ds(base_r, RS)], out_hbm.at[c, pl.ds(base_r, RS)])


@functools.partial(
    pl.kernel,
    out_type=(
        jax.ShapeDtypeStruct((EPAD, H), jnp.float32),
        jax.ShapeDtypeStruct((EPAD, H), jnp.float32),
    ),
    mesh=_mesh,
    compiler_params=pltpu.CompilerParams(needs_layout_passes=False, use_tc_tiling_on_sc=False),
    scratch_types=[
        pltpu.VMEM((CHUNK,), jnp.int32),
        pltpu.VMEM((CHUNK,), jnp.int32),
        pltpu.VMEM((CHUNK, H), jnp.float32),
        pltpu.VMEM((CHUNK, H), jnp.float32),
    ],
)
def _sc_edge_gather(a_hbm, b_hbm, src_hbm, dst_hbm, as_hbm, bd_hbm,
                    sidx, didx, rowsa, rowsb):
    c = lax.axis_index("c")
    s = lax.axis_index("s")
    w = c * NS + s
    base_e = w * EPW

    def body(i, carry):
        off = base_e + i * CHUNK
        pltpu.sync_copy(src_hbm.at[pl.ds(off, CHUNK)], sidx)
        pltpu.sync_copy(dst_hbm.at[pl.ds(off, CHUNK)], didx)
        pltpu.sync_copy(a_hbm.at[sidx], rowsa)
        pltpu.sync_copy(rowsa, as_hbm.at[pl.ds(off, CHUNK)])
        pltpu.sync_copy(b_hbm.at[didx], rowsb)
        pltpu.sync_copy(rowsb, bd_hbm.at[pl.ds(off, CHUNK)])
        return carry

    lax.fori_loop(0, NCHUNK, body, 0)


def _enc_body(xb, wb, bb, ob):
    ob[...] = jnp.dot(xb[...], wb[...], preferred_element_type=jnp.float32) + bb[...]


def _tc_encoder(xp, W_enc, b_enc):
    return pl.pallas_call(
        _enc_body,
        grid=(NP // BN,),
        in_specs=[
            pl.BlockSpec((BN, F_IN), lambda i: (i, 0)),
            pl.BlockSpec((F_IN, H), lambda i: (0, 0)),
            pl.BlockSpec((1, H), lambda i: (0, 0)),
        ],
        out_specs=pl.BlockSpec((BN, H), lambda i: (i, 0)),
        out_shape=jax.ShapeDtypeStruct((NP, H), jnp.float32),
    )(xp, W_enc, b_enc.reshape(1, H))


def _l0_body(degp, h0b, wb, dis_o, g_o):
    d = jnp.sum(degp[...], axis=0) + 1.0
    dis = lax.rsqrt(d)
    dis_o[...] = dis
    g_o[...] = jnp.dot(h0b[...], wb[...], preferred_element_type=jnp.float32) * dis


def _tc_layer0(degp, h0, W0):
    return pl.pallas_call(
        _l0_body,
        grid=(NP // BN,),
        in_specs=[
            pl.BlockSpec((NW, BN, 1), lambda i: (0, i, 0)),
            pl.BlockSpec((BN, H), lambda i: (i, 0)),
            pl.BlockSpec((H, H), lambda i: (0, 0)),
        ],
        out_specs=[
            pl.BlockSpec((BN, 1), lambda i: (i, 0)),
            pl.BlockSpec((BN, H), lambda i: (i, 0)),
        ],
        out_shape=[
            jax.ShapeDtypeStruct((NP, 1), jnp.float32),
            jax.ShapeDtypeStruct((NP, H), jnp.float32),
        ],
    )(degp, h0, W0)


def _layer_body(accp, gb, disb, bb, wnb, gn_o):
    dis = disb[...]
    h = jnp.maximum(dis * (jnp.sum(accp[...], axis=0) + gb[...]) + bb[...], 0.0)
    gn_o[...] = jnp.dot(h, wnb[...], preferred_element_type=jnp.float32) * dis


def _tc_layer(accp, g, dis, b, Wn):
    return pl.pallas_call(
        _layer_body,
        grid=(NP // BN,),
        in_specs=[
            pl.BlockSpec((NC, BN, H), lambda i: (0, i, 0)),
            pl.BlockSpec((BN, H), lambda i: (i, 0)),
            pl.BlockSpec((BN, 1), lambda i: (i, 0)),
            pl.BlockSpec((1, H), lambda i: (0, 0)),
            pl.BlockSpec((H, H), lambda i: (0, 0)),
        ],
        out_specs=pl.BlockSpec((BN, H), lambda i: (i, 0)),
        out_shape=jax.ShapeDtypeStruct((NP, H), jnp.float32),
    )(accp, g, dis, b, Wn)


def _head_body(accp, gb, disb, bb, wab, wbb, a_o, b_o):
    dis = disb[...]
    h = jnp.maximum(dis * (jnp.sum(accp[...], axis=0) + gb[...]) + bb[...], 0.0)
    a_o[...] = jnp.dot(h, wab[...], preferred_element_type=jnp.float32)
    b_o[...] = jnp.dot(h, wbb[...], preferred_element_type=jnp.float32)


def _tc_head(accp, g, dis, b, Wa, Wb):
    return pl.pallas_call(
        _head_body,
        grid=(NP // BN,),
        in_specs=[
            pl.BlockSpec((NC, BN, H), lambda i: (0, i, 0)),
            pl.BlockSpec((BN, H), lambda i: (i, 0)),
            pl.BlockSpec((BN, 1), lambda i: (i, 0)),
            pl.BlockSpec((1, H), lambda i: (0, 0)),
            pl.BlockSpec((H, H), lambda i: (0, 0)),
            pl.BlockSpec((H, H), lambda i: (0, 0)),
        ],
        out_specs=[
            pl.BlockSpec((BN, H), lambda i: (i, 0)),
            pl.BlockSpec((BN, H), lambda i: (i, 0)),
        ],
        out_shape=[
            jax.ShapeDtypeStruct((NP, H), jnp.float32),
            jax.ShapeDtypeStruct((NP, H), jnp.float32),
        ],
    )(accp, g, dis, b, Wa, Wb)


def _mlp_body(asb, bdb, eab, wcb, b0b, w1b, b1b, w2b, b2b, ob):
    z = asb[...] + bdb[...] + jnp.dot(
        eab[...], wcb[...], preferred_element_type=jnp.float32) + b0b[...]
    z = jnp.maximum(z, 0.0)
    y = jnp.maximum(
        jnp.dot(z, w1b[...], preferred_element_type=jnp.float32) + b1b[...], 0.0)
    ob[...] = jnp.dot(y, w2b[...], preferred_element_type=jnp.float32) + b2b[...]


def _tc_mlp(As, Bd, eap, Wc, b0, W1, b1, W2, b2):
    return pl.pallas_call(
        _mlp_body,
        grid=(EPAD // BE,),
        in_specs=[
            pl.BlockSpec((BE, H), lambda i: (i, 0)),
            pl.BlockSpec((BE, H), lambda i: (i, 0)),
            pl.BlockSpec((BE, F_E), lambda i: (i, 0)),
            pl.BlockSpec((F_E, H), lambda i: (0, 0)),
            pl.BlockSpec((1, H), lambda i: (0, 0)),
            pl.BlockSpec((H, H // 2), lambda i: (0, 0)),
            pl.BlockSpec((1, H // 2), lambda i: (0, 0)),
            pl.BlockSpec((H // 2, C), lambda i: (0, 0)),
            pl.BlockSpec((1, C), lambda i: (0, 0)),
        ],
        out_specs=pl.BlockSpec((BE, C), lambda i: (i, 0)),
        out_shape=jax.ShapeDtypeStruct((EPAD, C), jnp.float32),
    )(As, Bd, eap, Wc, b0, W1, b1, W2, b2)


def kernel(x, edge_index, edge_attr, W_enc, b_enc, W_c0, b_c0, W_c1, b_c1,
           W_c2, b_c2, W_p0, b_p0, W_p1, b_p1, W_p2, b_p2):
    src = edge_index[0]
    dst = edge_index[1]
    pad_e = EPAD - E
    srcp = jnp.concatenate([src, jnp.full((pad_e,), N, jnp.int32)])
    dstp = jnp.concatenate([dst, jnp.full((pad_e,), N, jnp.int32)])
    eap = jnp.pad(edge_attr, ((0, pad_e), (0, 0)))
    xp = jnp.pad(x, ((0, NP - N), (0, 0)))

    h0 = _tc_encoder(xp, W_enc, b_enc)
    degp = _sc_degree(dstp).reshape(NW, NP, 1)
    dis, g = _tc_layer0(degp, h0, W_c0)

    accp = _sc_conv(g, srcp, dstp)
    g = _tc_layer(accp, g, dis, b_c0.reshape(1, H), W_c1)
    accp = _sc_conv(g, srcp, dstp)
    g = _tc_layer(accp, g, dis, b_c1.reshape(1, H), W_c2)
    accp = _sc_conv(g, srcp, dstp)
    A, B = _tc_head(accp, g, dis, b_c2.reshape(1, H), W_p0[:H], W_p0[H:2 * H])

    As, Bd = _sc_edge_gather(A, B, srcp, dstp)
    out = _tc_mlp(As, Bd, eap, W_p0[2 * H:], b_p0.reshape(1, H),
                  W_p1, b_p1.reshape(1, H // 2), W_p2, b_p2.reshape(1, C))
    return out[:E]


# speedup vs baseline: 6.2017x; 1.3330x over previous
import functools

import jax
import jax.numpy as jnp
from jax import lax
from jax.experimental import pallas as pl
from jax.experimental.pallas import tpu as pltpu
from jax.experimental.pallas import tpu_sc as plsc

N = 10000
E = 320000
F_IN = 128
F_E = 16
H = 64
C = 10

NC = 2
NS = 16
NW = NC * NS

NP = 10240
EPAD = 327680
EPW = EPAD // NW
CHUNK = 128
NCHUNK = EPW // CHUNK
RS = NP // NS

BN = 1024
BE = 2048

_mesh = plsc.VectorSubcoreMesh(
    core_axis_name="c", subcore_axis_name="s", num_cores=NC, num_subcores=NS
)


def _zero2d(ref, nrows, ncols):
    z16 = jnp.zeros((16,), jnp.float32)

    def body(i, carry):
        r = i // (ncols // 16)
        j = (i % (ncols // 16)) * 16
        ref[r, pl.ds(j, 16)] = z16
        return carry

    lax.fori_loop(0, nrows * (ncols // 16), body, 0)


@functools.partial(
    pl.kernel,
    out_type=jax.ShapeDtypeStruct((NW, NP), jnp.float32),
    mesh=_mesh,
    compiler_params=pltpu.CompilerParams(needs_layout_passes=False, use_tc_tiling_on_sc=False),
    scratch_types=[
        pltpu.VMEM((NP,), jnp.float32),
        pltpu.VMEM((NCHUNK, CHUNK), jnp.int32),
    ],
)
def _sc_degree(dst_hbm, out_hbm, degv, didx):
    c = lax.axis_index("c")
    s = lax.axis_index("s")
    w = c * NS + s
    z16 = jnp.zeros((16,), jnp.float32)

    def zb(i, carry):
        degv[pl.ds(i * 16, 16)] = z16
        return carry

    lax.fori_loop(0, NP // 16, zb, 0)

    pltpu.sync_copy(dst_hbm.at[pl.ds(w * NCHUNK, NCHUNK)], didx)
    ones = jnp.ones((16,), jnp.float32)

    def body(i, carry):
        idx = didx[i // (CHUNK // 16), pl.ds((i % (CHUNK // 16)) * 16, 16)]
        plsc.addupdate_scatter(degv, [idx], ones)
        return carry

    lax.fori_loop(0, EPW // 16, body, 0)
    pltpu.sync_copy(degv, out_hbm.at[w])


@functools.partial(
    pl.kernel,
    out_type=jax.ShapeDtypeStruct((NC, NP, H), jnp.float32),
    mesh=_mesh,
    compiler_params=pltpu.CompilerParams(needs_layout_passes=False, use_tc_tiling_on_sc=False),
    scratch_types=[
        pltpu.VMEM((NCHUNK, CHUNK), jnp.int32),
        pltpu.VMEM((NCHUNK, CHUNK), jnp.int32),
        pltpu.VMEM((CHUNK, H), jnp.float32),
        pltpu.VMEM((CHUNK, H), jnp.float32),
        pltpu.VMEM((CHUNK, H), jnp.float32),
        pltpu.VMEM_SHARED((NP, H), jnp.float32),
        pltpu.SemaphoreType.DMA,
        pltpu.SemaphoreType.DMA,
    ],
)
def _sc_conv(g_hbm, src_hbm, dst_hbm, out_hbm, sidx2, didx2, rows0, rows1,
             zbuf, acc, gsem0, gsem1):
    c = lax.axis_index("c")
    s = lax.axis_index("s")
    w = c * NS + s
    _zero2d(zbuf, CHUNK, H)
    base_r = s * RS
    for k in range(RS // CHUNK):
        pltpu.sync_copy(zbuf, acc.at[pl.ds(base_r + k * CHUNK, CHUNK)])
    row0 = w * NCHUNK
    pltpu.sync_copy(src_hbm.at[pl.ds(row0, NCHUNK)], sidx2)
    pltpu.sync_copy(dst_hbm.at[pl.ds(row0, NCHUNK)], didx2)
    plsc.subcore_barrier()

    pltpu.async_copy(g_hbm.at[sidx2.at[0]], rows0, gsem0)

    def body(j, carry):
        i0 = 2 * j
        pltpu.async_copy(g_hbm.at[sidx2.at[i0 + 1]], rows1, gsem1)
        pltpu.make_async_copy(g_hbm.at[sidx2.at[i0]], rows0, gsem0).wait()
        pltpu.sync_copy(rows0, acc.at[didx2.at[i0]], add=True)

        @pl.when(j < NCHUNK // 2 - 1)
        def _():
            pltpu.async_copy(g_hbm.at[sidx2.at[i0 + 2]], rows0, gsem0)

        pltpu.make_async_copy(g_hbm.at[sidx2.at[i0 + 1]], rows1, gsem1).wait()
        pltpu.sync_copy(rows1, acc.at[didx2.at[i0 + 1]], add=True)
        return carry

    lax.fori_loop(0, NCHUNK // 2, body, 0)
    plsc.subcore_barrier()
    pltpu.sync_copy(acc.at[pl.ds(base_r, RS)], out_hbm.at[c, pl.ds(base_r, RS)])


@functools.partial(
    pl.kernel,
    out_type=(
        jax.ShapeDtypeStruct((EPAD, H), jnp.float32),
        jax.ShapeDtypeStruct((EPAD, H), jnp.float32),
    ),
    mesh=_mesh,
    compiler_params=pltpu.CompilerParams(needs_layout_passes=False, use_tc_tiling_on_sc=False),
    scratch_types=[
        pltpu.VMEM((NCHUNK, CHUNK), jnp.int32),
        pltpu.VMEM((NCHUNK, CHUNK), jnp.int32),
        pltpu.VMEM((CHUNK, H), jnp.float32),
        pltpu.VMEM((CHUNK, H), jnp.float32),
        pltpu.VMEM((CHUNK, H), jnp.float32),
        pltpu.VMEM((CHUNK, H), jnp.float32),
        pltpu.SemaphoreType.DMA,
        pltpu.SemaphoreType.DMA,
        pltpu.SemaphoreType.DMA,
        pltpu.SemaphoreType.DMA,
        pltpu.SemaphoreType.DMA,
        pltpu.SemaphoreType.DMA,
        pltpu.SemaphoreType.DMA,
        pltpu.SemaphoreType.DMA,
    ],
)
def _sc_edge_gather(a_hbm, b_hbm, src_hbm, dst_hbm, as_hbm, bd_hbm,
                    sidx2, didx2, bufa0, bufa1, bufb0, bufb1,
                    ga0, ga1, gb0, gb1, wa0, wa1, wb0, wb1):
    c = lax.axis_index("c")
    s = lax.axis_index("s")
    w = c * NS + s
    row0 = w * NCHUNK
    pltpu.sync_copy(src_hbm.at[pl.ds(row0, NCHUNK)], sidx2)
    pltpu.sync_copy(dst_hbm.at[pl.ds(row0, NCHUNK)], didx2)

    pltpu.async_copy(a_hbm.at[sidx2.at[0]], bufa0, ga0)
    pltpu.async_copy(b_hbm.at[didx2.at[0]], bufb0, gb0)
    nj = NCHUNK // 2

    def body(j, carry):
        i0 = 2 * j
        off0 = (row0 + i0) * CHUNK

        @pl.when(j > 0)
        def _():
            pltpu.make_async_copy(
                bufa1, as_hbm.at[pl.ds(off0 - CHUNK, CHUNK)], wa1).wait()
            pltpu.make_async_copy(
                bufb1, bd_hbm.at[pl.ds(off0 - CHUNK, CHUNK)], wb1).wait()
        pltpu.async_copy(a_hbm.at[sidx2.at[i0 + 1]], bufa1, ga1)
        pltpu.async_copy(b_hbm.at[didx2.at[i0 + 1]], bufb1, gb1)

        pltpu.make_async_copy(a_hbm.at[sidx2.at[i0]], bufa0, ga0).wait()
        pltpu.async_copy(bufa0, as_hbm.at[pl.ds(off0, CHUNK)], wa0)
        pltpu.make_async_copy(b_hbm.at[didx2.at[i0]], bufb0, gb0).wait()
        pltpu.async_copy(bufb0, bd_hbm.at[pl.ds(off0, CHUNK)], wb0)

        @pl.when(j < nj - 1)
        def _():
            pltpu.make_async_copy(
                bufa0, as_hbm.at[pl.ds(off0, CHUNK)], wa0).wait()
            pltpu.make_async_copy(
                bufb0, bd_hbm.at[pl.ds(off0, CHUNK)], wb0).wait()
            pltpu.async_copy(a_hbm.at[sidx2.at[i0 + 2]], bufa0, ga0)
            pltpu.async_copy(b_hbm.at[didx2.at[i0 + 2]], bufb0, gb0)

        pltpu.make_async_copy(a_hbm.at[sidx2.at[i0 + 1]], bufa1, ga1).wait()
        pltpu.async_copy(bufa1, as_hbm.at[pl.ds(off0 + CHUNK, CHUNK)], wa1)
        pltpu.make_async_copy(b_hbm.at[didx2.at[i0 + 1]], bufb1, gb1).wait()
        pltpu.async_copy(bufb1, bd_hbm.at[pl.ds(off0 + CHUNK, CHUNK)], wb1)
        return carry

    lax.fori_loop(0, nj, body, 0)
    last = row0 * CHUNK + (NCHUNK - 2) * CHUNK
    pltpu.make_async_copy(bufa0, as_hbm.at[pl.ds(last, CHUNK)], wa0).wait()
    pltpu.make_async_copy(bufb0, bd_hbm.at[pl.ds(last, CHUNK)], wb0).wait()
    pltpu.make_async_copy(bufa1, as_hbm.at[pl.ds(last + CHUNK, CHUNK)], wa1).wait()
    pltpu.make_async_copy(bufb1, bd_hbm.at[pl.ds(last + CHUNK, CHUNK)], wb1).wait()


def _enc_body(xb, wb, bb, ob):
    ob[...] = jnp.dot(xb[...], wb[...], preferred_element_type=jnp.float32) + bb[...]


def _tc_encoder(xp, W_enc, b_enc):
    return pl.pallas_call(
        _enc_body,
        grid=(NP // BN,),
        in_specs=[
            pl.BlockSpec((BN, F_IN), lambda i: (i, 0)),
            pl.BlockSpec((F_IN, H), lambda i: (0, 0)),
            pl.BlockSpec((1, H), lambda i: (0, 0)),
        ],
        out_specs=pl.BlockSpec((BN, H), lambda i: (i, 0)),
        out_shape=jax.ShapeDtypeStruct((NP, H), jnp.float32),
    )(xp, W_enc, b_enc.reshape(1, H))


def _l0_body(degp, h0b, wb, dis_o, g_o):
    d = jnp.sum(degp[...], axis=0) + 1.0
    dis = lax.rsqrt(d)
    dis_o[...] = dis
    g_o[...] = jnp.dot(h0b[...], wb[...], preferred_element_type=jnp.float32) * dis


def _tc_layer0(degp, h0, W0):
    return pl.pallas_call(
        _l0_body,
        grid=(NP // BN,),
        in_specs=[
            pl.BlockSpec((NW, BN, 1), lambda i: (0, i, 0)),
            pl.BlockSpec((BN, H), lambda i: (i, 0)),
            pl.BlockSpec((H, H), lambda i: (0, 0)),
        ],
        out_specs=[
            pl.BlockSpec((BN, 1), lambda i: (i, 0)),
            pl.BlockSpec((BN, H), lambda i: (i, 0)),
        ],
        out_shape=[
            jax.ShapeDtypeStruct((NP, 1), jnp.float32),
            jax.ShapeDtypeStruct((NP, H), jnp.float32),
        ],
    )(degp, h0, W0)


def _layer_body(accp, gb, disb, bb, wnb, gn_o):
    dis = disb[...]
    h = jnp.maximum(dis * (jnp.sum(accp[...], axis=0) + gb[...]) + bb[...], 0.0)
    gn_o[...] = jnp.dot(h, wnb[...], preferred_element_type=jnp.float32) * dis


def _tc_layer(accp, g, dis, b, Wn):
    return pl.pallas_call(
        _layer_body,
        grid=(NP // BN,),
        in_specs=[
            pl.BlockSpec((NC, BN, H), lambda i: (0, i, 0)),
            pl.BlockSpec((BN, H), lambda i: (i, 0)),
            pl.BlockSpec((BN, 1), lambda i: (i, 0)),
            pl.BlockSpec((1, H), lambda i: (0, 0)),
            pl.BlockSpec((H, H), lambda i: (0, 0)),
        ],
        out_specs=pl.BlockSpec((BN, H), lambda i: (i, 0)),
        out_shape=jax.ShapeDtypeStruct((NP, H), jnp.float32),
    )(accp, g, dis, b, Wn)


def _head_body(accp, gb, disb, bb, wab, wbb, a_o, b_o):
    dis = disb[...]
    h = jnp.maximum(dis * (jnp.sum(accp[...], axis=0) + gb[...]) + bb[...], 0.0)
    a_o[...] = jnp.dot(h, wab[...], preferred_element_type=jnp.float32)
    b_o[...] = jnp.dot(h, wbb[...], preferred_element_type=jnp.float32)


def _tc_head(accp, g, dis, b, Wa, Wb):
    return pl.pallas_call(
        _head_body,
        grid=(NP // BN,),
        in_specs=[
            pl.BlockSpec((NC, BN, H), lambda i: (0, i, 0)),
            pl.BlockSpec((BN, H), lambda i: (i, 0)),
            pl.BlockSpec((BN, 1), lambda i: (i, 0)),
            pl.BlockSpec((1, H), lambda i: (0, 0)),
            pl.BlockSpec((H, H), lambda i: (0, 0)),
            pl.BlockSpec((H, H), lambda i: (0, 0)),
        ],
        out_specs=[
            pl.BlockSpec((BN, H), lambda i: (i, 0)),
            pl.BlockSpec((BN, H), lambda i: (i, 0)),
        ],
        out_shape=[
            jax.ShapeDtypeStruct((NP, H), jnp.float32),
            jax.ShapeDtypeStruct((NP, H), jnp.float32),
        ],
    )(accp, g, dis, b, Wa, Wb)


def _mlp_body(asb, bdb, eab, wcb, b0b, w1b, b1b, w2b, b2b, ob):
    z = asb[...] + bdb[...] + jnp.dot(
        eab[...], wcb[...], preferred_element_type=jnp.float32) + b0b[...]
    z = jnp.maximum(z, 0.0)
    y = jnp.maximum(
        jnp.dot(z, w1b[...], preferred_element_type=jnp.float32) + b1b[...], 0.0)
    ob[...] = jnp.dot(y, w2b[...], preferred_element_type=jnp.float32) + b2b[...]


def _tc_mlp(As, Bd, eap, Wc, b0, W1, b1, W2, b2):
    return pl.pallas_call(
        _mlp_body,
        grid=(EPAD // BE,),
        in_specs=[
            pl.BlockSpec((BE, H), lambda i: (i, 0)),
            pl.BlockSpec((BE, H), lambda i: (i, 0)),
            pl.BlockSpec((BE, F_E), lambda i: (i, 0)),
            pl.BlockSpec((F_E, H), lambda i: (0, 0)),
            pl.BlockSpec((1, H), lambda i: (0, 0)),
            pl.BlockSpec((H, H // 2), lambda i: (0, 0)),
            pl.BlockSpec((1, H // 2), lambda i: (0, 0)),
            pl.BlockSpec((H // 2, C), lambda i: (0, 0)),
            pl.BlockSpec((1, C), lambda i: (0, 0)),
        ],
        out_specs=pl.BlockSpec((BE, C), lambda i: (i, 0)),
        out_shape=jax.ShapeDtypeStruct((EPAD, C), jnp.float32),
    )(As, Bd, eap, Wc, b0, W1, b1, W2, b2)


def kernel(x, edge_index, edge_attr, W_enc, b_enc, W_c0, b_c0, W_c1, b_c1,
           W_c2, b_c2, W_p0, b_p0, W_p1, b_p1, W_p2, b_p2):
    src = edge_index[0]
    dst = edge_index[1]
    pad_e = EPAD - E
    srcp = jnp.concatenate([src, jnp.full((pad_e,), N, jnp.int32)]).reshape(-1, CHUNK)
    dstp = jnp.concatenate([dst, jnp.full((pad_e,), N, jnp.int32)]).reshape(-1, CHUNK)
    eap = jnp.pad(edge_attr, ((0, pad_e), (0, 0)))
    xp = jnp.pad(x, ((0, NP - N), (0, 0)))

    h0 = _tc_encoder(xp, W_enc, b_enc)
    degp = _sc_degree(dstp).reshape(NW, NP, 1)
    dis, g = _tc_layer0(degp, h0, W_c0)

    accp = _sc_conv(g, srcp, dstp)
    g = _tc_layer(accp, g, dis, b_c0.reshape(1, H), W_c1)
    accp = _sc_conv(g, srcp, dstp)
    g = _tc_layer(accp, g, dis, b_c1.reshape(1, H), W_c2)
    accp = _sc_conv(g, srcp, dstp)
    A, B = _tc_head(accp, g, dis, b_c2.reshape(1, H), W_p0[:H], W_p0[H:2 * H])

    As, Bd = _sc_edge_gather(A, B, srcp, dstp)
    out = _tc_mlp(As, Bd, eap, W_p0[2 * H:], b_p0.reshape(1, H),
                  W_p1, b_p1.reshape(1, H // 2), W_p2, b_p2.reshape(1, C))
    return out[:E]


# speedup vs baseline: 6.2905x; 1.0143x over previous
import functools

import jax
import jax.numpy as jnp
from jax import lax
from jax.experimental import pallas as pl
from jax.experimental.pallas import tpu as pltpu
from jax.experimental.pallas import tpu_sc as plsc

N = 10000
E = 320000
F_IN = 128
F_E = 16
H = 64
C = 10

NC = 2
NS = 16
NW = NC * NS

NP = 10240
EPAD = 327680
EPW = EPAD // NW
CHUNK = 128
NCHUNK = EPW // CHUNK
RS = NP // NS

BN = 1024
BE = 2048

_mesh = plsc.VectorSubcoreMesh(
    core_axis_name="c", subcore_axis_name="s", num_cores=NC, num_subcores=NS
)


def _zero2d(ref, nrows, ncols):
    z16 = jnp.zeros((16,), jnp.float32)

    def body(i, carry):
        r = i // (ncols // 16)
        j = (i % (ncols // 16)) * 16
        ref[r, pl.ds(j, 16)] = z16
        return carry

    lax.fori_loop(0, nrows * (ncols // 16), body, 0)


@functools.partial(
    pl.kernel,
    out_type=jax.ShapeDtypeStruct((NW, NP), jnp.float32),
    mesh=_mesh,
    compiler_params=pltpu.CompilerParams(needs_layout_passes=False, use_tc_tiling_on_sc=False),
    scratch_types=[
        pltpu.VMEM((NP,), jnp.float32),
        pltpu.VMEM((NCHUNK, CHUNK), jnp.int32),
    ],
)
def _sc_degree(dst_hbm, out_hbm, degv, didx):
    c = lax.axis_index("c")
    s = lax.axis_index("s")
    w = c * NS + s
    z16 = jnp.zeros((16,), jnp.float32)

    def zb(i, carry):
        degv[pl.ds(i * 16, 16)] = z16
        return carry

    lax.fori_loop(0, NP // 16, zb, 0)

    pltpu.sync_copy(dst_hbm.at[pl.ds(w * NCHUNK, NCHUNK)], didx)
    ones = jnp.ones((16,), jnp.float32)

    def body(i, carry):
        idx = didx[i // (CHUNK // 16), pl.ds((i % (CHUNK // 16)) * 16, 16)]
        plsc.addupdate_scatter(degv, [idx], ones)
        return carry

    lax.fori_loop(0, EPW // 16, body, 0)
    pltpu.sync_copy(degv, out_hbm.at[w])


@functools.partial(
    pl.kernel,
    out_type=jax.ShapeDtypeStruct((NC, NP, H), jnp.float32),
    mesh=_mesh,
    compiler_params=pltpu.CompilerParams(needs_layout_passes=False, use_tc_tiling_on_sc=False),
    scratch_types=[
        pltpu.VMEM((NCHUNK, CHUNK), jnp.int32),
        pltpu.VMEM((NCHUNK, CHUNK), jnp.int32),
        pltpu.VMEM((CHUNK, H), jnp.float32),
        pltpu.VMEM((CHUNK, H), jnp.float32),
        pltpu.VMEM((CHUNK, H), jnp.float32),
        pltpu.VMEM_SHARED((NP, H), jnp.float32),
        pltpu.SemaphoreType.DMA,
        pltpu.SemaphoreType.DMA,
    ],
)
def _sc_conv(g_hbm, src_hbm, dst_hbm, out_hbm, sidx2, didx2, rows0, rows1,
             zbuf, acc, gsem0, gsem1):
    c = lax.axis_index("c")
    s = lax.axis_index("s")
    w = c * NS + s
    _zero2d(zbuf, CHUNK, H)
    base_r = s * RS
    for k in range(RS // CHUNK):
        pltpu.sync_copy(zbuf, acc.at[pl.ds(base_r + k * CHUNK, CHUNK)])
    row0 = w * NCHUNK
    pltpu.sync_copy(src_hbm.at[pl.ds(row0, NCHUNK)], sidx2)
    pltpu.sync_copy(dst_hbm.at[pl.ds(row0, NCHUNK)], didx2)
    plsc.subcore_barrier()

    pltpu.async_copy(g_hbm.at[sidx2.at[0]], rows0, gsem0)

    def body(j, carry):
        i0 = 2 * j
        pltpu.async_copy(g_hbm.at[sidx2.at[i0 + 1]], rows1, gsem1)
        pltpu.make_async_copy(g_hbm.at[sidx2.at[i0]], rows0, gsem0).wait()
        pltpu.sync_copy(rows0, acc.at[didx2.at[i0]], add=True)

        @pl.when(j < NCHUNK // 2 - 1)
        def _():
            pltpu.async_copy(g_hbm.at[sidx2.at[i0 + 2]], rows0, gsem0)

        pltpu.make_async_copy(g_hbm.at[sidx2.at[i0 + 1]], rows1, gsem1).wait()
        pltpu.sync_copy(rows1, acc.at[didx2.at[i0 + 1]], add=True)
        return carry

    lax.fori_loop(0, NCHUNK // 2, body, 0)
    plsc.subcore_barrier()
    pltpu.sync_copy(acc.at[pl.ds(base_r, RS)], out_hbm.at[c, pl.ds(base_r, RS)])


@functools.partial(
    pl.kernel,
    out_type=jax.ShapeDtypeStruct((EPAD, 2 * H), jnp.float32),
    mesh=_mesh,
    compiler_params=pltpu.CompilerParams(needs_layout_passes=False, use_tc_tiling_on_sc=False),
    scratch_types=[
        pltpu.VMEM((NCHUNK, CHUNK), jnp.int32),
        pltpu.VMEM((NCHUNK, CHUNK), jnp.int32),
        pltpu.VMEM((CHUNK, H), jnp.float32),
        pltpu.VMEM((CHUNK, H), jnp.float32),
        pltpu.VMEM((CHUNK, H), jnp.float32),
        pltpu.VMEM((CHUNK, H), jnp.float32),
        pltpu.SemaphoreType.DMA,
        pltpu.SemaphoreType.DMA,
        pltpu.SemaphoreType.DMA,
        pltpu.SemaphoreType.DMA,
        pltpu.SemaphoreType.DMA,
        pltpu.SemaphoreType.DMA,
        pltpu.SemaphoreType.DMA,
        pltpu.SemaphoreType.DMA,
    ],
)
def _sc_edge_gather(h_hbm, src_hbm, dst_hbm, gh_hbm,
                    sidx2, didx2, bufa0, bufa1, bufb0, bufb1,
                    ga0, ga1, gb0, gb1, wa0, wa1, wb0, wb1):
    c = lax.axis_index("c")
    s = lax.axis_index("s")
    w = c * NS + s
    row0 = w * NCHUNK
    pltpu.sync_copy(src_hbm.at[pl.ds(row0, NCHUNK)], sidx2)
    pltpu.sync_copy(dst_hbm.at[pl.ds(row0, NCHUNK)], didx2)

    def gather(i, bufa, bufb, sa, sb):
        pltpu.async_copy(h_hbm.at[sidx2.at[i]], bufa, sa)
        pltpu.async_copy(h_hbm.at[didx2.at[i]], bufb, sb)

    def gwait(i, bufa, bufb, sa, sb):
        pltpu.make_async_copy(h_hbm.at[sidx2.at[i]], bufa, sa).wait()
        pltpu.make_async_copy(h_hbm.at[didx2.at[i]], bufb, sb).wait()

    def dst_l(off):
        return gh_hbm.at[pl.ds(off, CHUNK), pl.ds(0, H)]

    def dst_r(off):
        return gh_hbm.at[pl.ds(off, CHUNK), pl.ds(H, H)]

    gather(0, bufa0, bufb0, ga0, gb0)
    nj = NCHUNK // 2

    def body(j, carry):
        i0 = 2 * j
        off0 = (row0 + i0) * CHUNK

        @pl.when(j > 0)
        def _():
            pltpu.make_async_copy(bufa1, dst_l(off0 - CHUNK), wa1).wait()
            pltpu.make_async_copy(bufb1, dst_r(off0 - CHUNK), wb1).wait()
        gather(i0 + 1, bufa1, bufb1, ga1, gb1)

        pltpu.make_async_copy(h_hbm.at[sidx2.at[i0]], bufa0, ga0).wait()
        pltpu.async_copy(bufa0, dst_l(off0), wa0)
        pltpu.make_async_copy(h_hbm.at[didx2.at[i0]], bufb0, gb0).wait()
        pltpu.async_copy(bufb0, dst_r(off0), wb0)

        @pl.when(j < nj - 1)
        def _():
            pltpu.make_async_copy(bufa0, dst_l(off0), wa0).wait()
            pltpu.make_async_copy(bufb0, dst_r(off0), wb0).wait()
            gather(i0 + 2, bufa0, bufb0, ga0, gb0)

        pltpu.make_async_copy(h_hbm.at[sidx2.at[i0 + 1]], bufa1, ga1).wait()
        pltpu.async_copy(bufa1, dst_l(off0 + CHUNK), wa1)
        pltpu.make_async_copy(h_hbm.at[didx2.at[i0 + 1]], bufb1, gb1).wait()
        pltpu.async_copy(bufb1, dst_r(off0 + CHUNK), wb1)
        return carry

    lax.fori_loop(0, nj, body, 0)
    last = row0 * CHUNK + (NCHUNK - 2) * CHUNK
    pltpu.make_async_copy(bufa0, dst_l(last), wa0).wait()
    pltpu.make_async_copy(bufb0, dst_r(last), wb0).wait()
    pltpu.make_async_copy(bufa1, dst_l(last + CHUNK), wa1).wait()
    pltpu.make_async_copy(bufb1, dst_r(last + CHUNK), wb1).wait()


def _enc_body(xb, wb, bb, ob):
    ob[...] = jnp.dot(xb[...], wb[...], preferred_element_type=jnp.float32) + bb[...]


def _tc_encoder(xp, W_enc, b_enc):
    return pl.pallas_call(
        _enc_body,
        grid=(NP // BN,),
        in_specs=[
            pl.BlockSpec((BN, F_IN), lambda i: (i, 0)),
            pl.BlockSpec((F_IN, H), lambda i: (0, 0)),
            pl.BlockSpec((1, H), lambda i: (0, 0)),
        ],
        out_specs=pl.BlockSpec((BN, H), lambda i: (i, 0)),
        out_shape=jax.ShapeDtypeStruct((NP, H), jnp.float32),
    )(xp, W_enc, b_enc.reshape(1, H))


def _l0_body(degp, h0b, wb, dis_o, g_o):
    d = jnp.sum(degp[...], axis=0) + 1.0
    dis = lax.rsqrt(d)
    dis_o[...] = dis
    g_o[...] = jnp.dot(h0b[...], wb[...], preferred_element_type=jnp.float32) * dis


def _tc_layer0(degp, h0, W0):
    return pl.pallas_call(
        _l0_body,
        grid=(NP // BN,),
        in_specs=[
            pl.BlockSpec((NW, BN, 1), lambda i: (0, i, 0)),
            pl.BlockSpec((BN, H), lambda i: (i, 0)),
            pl.BlockSpec((H, H), lambda i: (0, 0)),
        ],
        out_specs=[
            pl.BlockSpec((BN, 1), lambda i: (i, 0)),
            pl.BlockSpec((BN, H), lambda i: (i, 0)),
        ],
        out_shape=[
            jax.ShapeDtypeStruct((NP, 1), jnp.float32),
            jax.ShapeDtypeStruct((NP, H), jnp.float32),
        ],
    )(degp, h0, W0)


def _layer_body(accp, gb, disb, bb, wnb, gn_o):
    dis = disb[...]
    h = jnp.maximum(dis * (jnp.sum(accp[...], axis=0) + gb[...]) + bb[...], 0.0)
    gn_o[...] = jnp.dot(h, wnb[...], preferred_element_type=jnp.float32) * dis


def _tc_layer(accp, g, dis, b, Wn):
    return pl.pallas_call(
        _layer_body,
        grid=(NP // BN,),
        in_specs=[
            pl.BlockSpec((NC, BN, H), lambda i: (0, i, 0)),
            pl.BlockSpec((BN, H), lambda i: (i, 0)),
            pl.BlockSpec((BN, 1), lambda i: (i, 0)),
            pl.BlockSpec((1, H), lambda i: (0, 0)),
            pl.BlockSpec((H, H), lambda i: (0, 0)),
        ],
        out_specs=pl.BlockSpec((BN, H), lambda i: (i, 0)),
        out_shape=jax.ShapeDtypeStruct((NP, H), jnp.float32),
    )(accp, g, dis, b, Wn)


def _h3_body(accp, gb, disb, bb, h_o):
    dis = disb[...]
    h_o[...] = jnp.maximum(
        dis * (jnp.sum(accp[...], axis=0) + gb[...]) + bb[...], 0.0)


def _tc_h3(accp, g, dis, b):
    return pl.pallas_call(
        _h3_body,
        grid=(NP // BN,),
        in_specs=[
            pl.BlockSpec((NC, BN, H), lambda i: (0, i, 0)),
            pl.BlockSpec((BN, H), lambda i: (i, 0)),
            pl.BlockSpec((BN, 1), lambda i: (i, 0)),
            pl.BlockSpec((1, H), lambda i: (0, 0)),
        ],
        out_specs=pl.BlockSpec((BN, H), lambda i: (i, 0)),
        out_shape=jax.ShapeDtypeStruct((NP, H), jnp.float32),
    )(accp, g, dis, b)


def _mlp_body(ghb, eab, w0b, wcb, b0b, w1b, b1b, w2b, b2b, ob):
    z = jnp.dot(ghb[...], w0b[...], preferred_element_type=jnp.float32) + jnp.dot(
        eab[...], wcb[...], preferred_element_type=jnp.float32) + b0b[...]
    z = jnp.maximum(z, 0.0)
    y = jnp.maximum(
        jnp.dot(z, w1b[...], preferred_element_type=jnp.float32) + b1b[...], 0.0)
    ob[...] = jnp.dot(y, w2b[...], preferred_element_type=jnp.float32) + b2b[...]


def _tc_mlp(GH, eap, W0, Wc, b0, W1, b1, W2, b2):
    return pl.pallas_call(
        _mlp_body,
        grid=(EPAD // BE,),
        in_specs=[
            pl.BlockSpec((BE, 2 * H), lambda i: (i, 0)),
            pl.BlockSpec((BE, F_E), lambda i: (i, 0)),
            pl.BlockSpec((2 * H, H), lambda i: (0, 0)),
            pl.BlockSpec((F_E, H), lambda i: (0, 0)),
            pl.BlockSpec((1, H), lambda i: (0, 0)),
            pl.BlockSpec((H, H // 2), lambda i: (0, 0)),
            pl.BlockSpec((1, H // 2), lambda i: (0, 0)),
            pl.BlockSpec((H // 2, C), lambda i: (0, 0)),
            pl.BlockSpec((1, C), lambda i: (0, 0)),
        ],
        out_specs=pl.BlockSpec((BE, C), lambda i: (i, 0)),
        out_shape=jax.ShapeDtypeStruct((EPAD, C), jnp.float32),
    )(GH, eap, W0, Wc, b0, W1, b1, W2, b2)


def kernel(x, edge_index, edge_attr, W_enc, b_enc, W_c0, b_c0, W_c1, b_c1,
           W_c2, b_c2, W_p0, b_p0, W_p1, b_p1, W_p2, b_p2):
    src = edge_index[0]
    dst = edge_index[1]
    pad_e = EPAD - E
    srcp = jnp.concatenate([src, jnp.full((pad_e,), N, jnp.int32)]).reshape(-1, CHUNK)
    dstp = jnp.concatenate([dst, jnp.full((pad_e,), N, jnp.int32)]).reshape(-1, CHUNK)
    eap = jnp.pad(edge_attr, ((0, pad_e), (0, 0)))
    xp = jnp.pad(x, ((0, NP - N), (0, 0)))

    h0 = _tc_encoder(xp, W_enc, b_enc)
    degp = _sc_degree(dstp).reshape(NW, NP, 1)
    dis, g = _tc_layer0(degp, h0, W_c0)

    accp = _sc_conv(g, srcp, dstp)
    g = _tc_layer(accp, g, dis, b_c0.reshape(1, H), W_c1)
    accp = _sc_conv(g, srcp, dstp)
    g = _tc_layer(accp, g, dis, b_c1.reshape(1, H), W_c2)
    accp = _sc_conv(g, srcp, dstp)
    h3 = _tc_h3(accp, g, dis, b_c2.reshape(1, H))

    GH = _sc_edge_gather(h3, srcp, dstp)
    out = _tc_mlp(GH, eap, W_p0[:2 * H], W_p0[2 * H:], b_p0.reshape(1, H),
                  W_p1, b_p1.reshape(1, H // 2), W_p2, b_p2.reshape(1, C))
    return out[:E]


# speedup vs baseline: 7.8979x; 1.2555x over previous
import functools

import jax
import jax.numpy as jnp
from jax import lax
from jax.experimental import pallas as pl
from jax.experimental.pallas import tpu as pltpu
from jax.experimental.pallas import tpu_sc as plsc

N = 10000
E = 320000
F_IN = 128
F_E = 16
H = 64
C = 10

NC = 2
NS = 16
NW = NC * NS

NP = 10240
EPAD = 327680
EPW = EPAD // NW
CHUNK = 128
NCHUNK = EPW // CHUNK
EROWS = EPAD // CHUNK
RS = NP // NS

KC0 = 112
KC1 = 48
KMAX = max(KC0, KC1)
IDXROWS = EROWS + KMAX

BN = 1024
BE = 2560

_mesh = plsc.VectorSubcoreMesh(
    core_axis_name="c", subcore_axis_name="s", num_cores=NC, num_subcores=NS
)


def _zero2d(ref, nrows, ncols):
    z16 = jnp.zeros((16,), jnp.float32)

    def body(i, carry):
        r = i // (ncols // 16)
        j = (i % (ncols // 16)) * 16
        ref[r, pl.ds(j, 16)] = z16
        return carry

    lax.fori_loop(0, nrows * (ncols // 16), body, 0)


@functools.partial(
    pl.kernel,
    out_type=jax.ShapeDtypeStruct((NW, NP), jnp.float32),
    mesh=_mesh,
    compiler_params=pltpu.CompilerParams(needs_layout_passes=False, use_tc_tiling_on_sc=False),
    scratch_types=[
        pltpu.VMEM((NP,), jnp.float32),
        pltpu.VMEM((NCHUNK, CHUNK), jnp.int32),
    ],
)
def _sc_degree(dst_hbm, out_hbm, degv, didx):
    c = lax.axis_index("c")
    s = lax.axis_index("s")
    w = c * NS + s
    z16 = jnp.zeros((16,), jnp.float32)

    def zb(i, carry):
        degv[pl.ds(i * 16, 16)] = z16
        return carry

    lax.fori_loop(0, NP // 16, zb, 0)

    pltpu.sync_copy(dst_hbm.at[pl.ds(w * NCHUNK, NCHUNK)], didx)
    ones = jnp.ones((16,), jnp.float32)

    def body(i, carry):
        idx = didx[i // (CHUNK // 16), pl.ds((i % (CHUNK // 16)) * 16, 16)]
        plsc.addupdate_scatter(degv, [idx], ones)
        return carry

    lax.fori_loop(0, EPW // 16, body, 0)
    pltpu.sync_copy(degv, out_hbm.at[w])


@functools.partial(
    pl.kernel,
    out_type=jax.ShapeDtypeStruct((NC, NP, H), jnp.float32),
    mesh=_mesh,
    compiler_params=pltpu.CompilerParams(needs_layout_passes=False, use_tc_tiling_on_sc=False),
    scratch_types=[
        pltpu.VMEM((KMAX, CHUNK), jnp.int32),
        pltpu.VMEM((KMAX, CHUNK), jnp.int32),
        pltpu.VMEM((CHUNK, H), jnp.float32),
        pltpu.VMEM((CHUNK, H), jnp.float32),
        pltpu.VMEM((CHUNK, H), jnp.float32),
        pltpu.VMEM_SHARED((NP, H), jnp.float32),
        pltpu.SemaphoreType.DMA,
        pltpu.SemaphoreType.DMA,
    ],
)
def _sc_conv(g_hbm, src_hbm, dst_hbm, out_hbm, sidx2, didx2, rows0, rows1,
             zbuf, acc, gsem0, gsem1):
    c = lax.axis_index("c")
    s = lax.axis_index("s")
    _zero2d(zbuf, CHUNK, H)
    base_r = s * RS
    for k in range(RS // CHUNK):
        pltpu.sync_copy(zbuf, acc.at[pl.ds(base_r + k * CHUNK, CHUNK)])
    row0 = jnp.where(c == 0, s * KC0, NS * KC0 + s * KC1)
    nj = jnp.where(c == 0, KC0 // 2, KC1 // 2)
    pltpu.sync_copy(src_hbm.at[pl.ds(row0, KMAX)], sidx2)
    pltpu.sync_copy(dst_hbm.at[pl.ds(row0, KMAX)], didx2)
    plsc.subcore_barrier()

    pltpu.async_copy(g_hbm.at[sidx2.at[0]], rows0, gsem0)

    def body(j, carry):
        i0 = 2 * j
        pltpu.async_copy(g_hbm.at[sidx2.at[i0 + 1]], rows1, gsem1)
        pltpu.make_async_copy(g_hbm.at[sidx2.at[i0]], rows0, gsem0).wait()
        pltpu.sync_copy(rows0, acc.at[didx2.at[i0]], add=True)

        @pl.when(j < nj - 1)
        def _():
            pltpu.async_copy(g_hbm.at[sidx2.at[i0 + 2]], rows0, gsem0)

        pltpu.make_async_copy(g_hbm.at[sidx2.at[i0 + 1]], rows1, gsem1).wait()
        pltpu.sync_copy(rows1, acc.at[didx2.at[i0 + 1]], add=True)
        return carry

    lax.fori_loop(0, nj, body, 0)
    plsc.subcore_barrier()
    pltpu.sync_copy(acc.at[pl.ds(base_r, RS)], out_hbm.at[c, pl.ds(base_r, RS)])


@functools.partial(
    pl.kernel,
    out_type=jax.ShapeDtypeStruct((EPAD, 2 * H), jnp.float32),
    mesh=_mesh,
    compiler_params=pltpu.CompilerParams(needs_layout_passes=False, use_tc_tiling_on_sc=False),
    scratch_types=[
        pltpu.VMEM((KMAX, CHUNK), jnp.int32),
        pltpu.VMEM((KMAX, CHUNK), jnp.int32),
        pltpu.VMEM((CHUNK, H), jnp.float32),
        pltpu.VMEM((CHUNK, H), jnp.float32),
        pltpu.VMEM((CHUNK, H), jnp.float32),
        pltpu.VMEM((CHUNK, H), jnp.float32),
        pltpu.SemaphoreType.DMA,
        pltpu.SemaphoreType.DMA,
        pltpu.SemaphoreType.DMA,
        pltpu.SemaphoreType.DMA,
        pltpu.SemaphoreType.DMA,
        pltpu.SemaphoreType.DMA,
        pltpu.SemaphoreType.DMA,
        pltpu.SemaphoreType.DMA,
    ],
)
def _sc_edge_gather(a_hbm, b_hbm, src_hbm, dst_hbm, gh_hbm,
                    sidx2, didx2, bufa0, bufa1, bufb0, bufb1,
                    ga0, ga1, gb0, gb1, wa0, wa1, wb0, wb1):
    c = lax.axis_index("c")
    s = lax.axis_index("s")
    row0 = jnp.where(c == 0, s * KC0, NS * KC0 + s * KC1)
    nj = jnp.where(c == 0, KC0 // 2, KC1 // 2)
    pltpu.sync_copy(src_hbm.at[pl.ds(row0, KMAX)], sidx2)
    pltpu.sync_copy(dst_hbm.at[pl.ds(row0, KMAX)], didx2)

    def gather(i, bufa, bufb, sa, sb):
        pltpu.async_copy(a_hbm.at[sidx2.at[i]], bufa, sa)
        pltpu.async_copy(b_hbm.at[didx2.at[i]], bufb, sb)

    def dst_l(off):
        return gh_hbm.at[pl.ds(off, CHUNK), pl.ds(0, H)]

    def dst_r(off):
        return gh_hbm.at[pl.ds(off, CHUNK), pl.ds(H, H)]

    gather(0, bufa0, bufb0, ga0, gb0)

    def body(j, carry):
        i0 = 2 * j
        off0 = (row0 + i0) * CHUNK

        @pl.when(j > 0)
        def _():
            pltpu.make_async_copy(bufa1, dst_l(off0 - CHUNK), wa1).wait()
            pltpu.make_async_copy(bufb1, dst_r(off0 - CHUNK), wb1).wait()
        gather(i0 + 1, bufa1, bufb1, ga1, gb1)

        pltpu.make_async_copy(a_hbm.at[sidx2.at[i0]], bufa0, ga0).wait()
        pltpu.async_copy(bufa0, dst_l(off0), wa0)
        pltpu.make_async_copy(b_hbm.at[didx2.at[i0]], bufb0, gb0).wait()
        pltpu.async_copy(bufb0, dst_r(off0), wb0)

        @pl.when(j < nj - 1)
        def _():
            pltpu.make_async_copy(bufa0, dst_l(off0), wa0).wait()
            pltpu.make_async_copy(bufb0, dst_r(off0), wb0).wait()
            gather(i0 + 2, bufa0, bufb0, ga0, gb0)

        pltpu.make_async_copy(a_hbm.at[sidx2.at[i0 + 1]], bufa1, ga1).wait()
        pltpu.async_copy(bufa1, dst_l(off0 + CHUNK), wa1)
        pltpu.make_async_copy(b_hbm.at[didx2.at[i0 + 1]], bufb1, gb1).wait()
        pltpu.async_copy(bufb1, dst_r(off0 + CHUNK), wb1)
        return carry

    lax.fori_loop(0, nj, body, 0)
    last = (row0 + 2 * nj - 2) * CHUNK
    pltpu.make_async_copy(bufa0, dst_l(last), wa0).wait()
    pltpu.make_async_copy(bufb0, dst_r(last), wb0).wait()
    pltpu.make_async_copy(bufa1, dst_l(last + CHUNK), wa1).wait()
    pltpu.make_async_copy(bufb1, dst_r(last + CHUNK), wb1).wait()


def _enc_body(xb, wb, bb, ob):
    ob[...] = jnp.dot(xb[...], wb[...], preferred_element_type=jnp.float32) + bb[...]


def _tc_encoder(xp, W_enc, b_enc):
    return pl.pallas_call(
        _enc_body,
        grid=(NP // BN,),
        in_specs=[
            pl.BlockSpec((BN, F_IN), lambda i: (i, 0)),
            pl.BlockSpec((F_IN, H), lambda i: (0, 0)),
            pl.BlockSpec((1, H), lambda i: (0, 0)),
        ],
        out_specs=pl.BlockSpec((BN, H), lambda i: (i, 0)),
        out_shape=jax.ShapeDtypeStruct((NP, H), jnp.float32),
    )(xp, W_enc, b_enc.reshape(1, H))


def _l0_body(degp, h0b, wb, dis_o, g_o):
    d = jnp.sum(degp[...], axis=0) + 1.0
    dis = lax.rsqrt(d)
    dis_o[...] = dis
    g_o[...] = jnp.dot(h0b[...], wb[...], preferred_element_type=jnp.float32) * dis


def _tc_layer0(degp, h0, W0):
    return pl.pallas_call(
        _l0_body,
        grid=(NP // BN,),
        in_specs=[
            pl.BlockSpec((NW, BN, 1), lambda i: (0, i, 0)),
            pl.BlockSpec((BN, H), lambda i: (i, 0)),
            pl.BlockSpec((H, H), lambda i: (0, 0)),
        ],
        out_specs=[
            pl.BlockSpec((BN, 1), lambda i: (i, 0)),
            pl.BlockSpec((BN, H), lambda i: (i, 0)),
        ],
        out_shape=[
            jax.ShapeDtypeStruct((NP, 1), jnp.float32),
            jax.ShapeDtypeStruct((NP, H), jnp.float32),
        ],
    )(degp, h0, W0)


def _layer_body(accp, gb, disb, bb, wnb, gn_o):
    dis = disb[...]
    h = jnp.maximum(dis * (jnp.sum(accp[...], axis=0) + gb[...]) + bb[...], 0.0)
    gn_o[...] = jnp.dot(h, wnb[...], preferred_element_type=jnp.float32) * dis


def _tc_layer(accp, g, dis, b, Wn):
    return pl.pallas_call(
        _layer_body,
        grid=(NP // BN,),
        in_specs=[
            pl.BlockSpec((NC, BN, H), lambda i: (0, i, 0)),
            pl.BlockSpec((BN, H), lambda i: (i, 0)),
            pl.BlockSpec((BN, 1), lambda i: (i, 0)),
            pl.BlockSpec((1, H), lambda i: (0, 0)),
            pl.BlockSpec((H, H), lambda i: (0, 0)),
        ],
        out_specs=pl.BlockSpec((BN, H), lambda i: (i, 0)),
        out_shape=jax.ShapeDtypeStruct((NP, H), jnp.float32),
    )(accp, g, dis, b, Wn)


def _head_body(accp, gb, disb, bb, wab, wbb, a_o, b_o):
    dis = disb[...]
    h = jnp.maximum(dis * (jnp.sum(accp[...], axis=0) + gb[...]) + bb[...], 0.0)
    a_o[...] = jnp.dot(h, wab[...], preferred_element_type=jnp.float32)
    b_o[...] = jnp.dot(h, wbb[...], preferred_element_type=jnp.float32)


def _tc_head(accp, g, dis, b, Wa, Wb):
    return pl.pallas_call(
        _head_body,
        grid=(NP // BN,),
        in_specs=[
            pl.BlockSpec((NC, BN, H), lambda i: (0, i, 0)),
            pl.BlockSpec((BN, H), lambda i: (i, 0)),
            pl.BlockSpec((BN, 1), lambda i: (i, 0)),
            pl.BlockSpec((1, H), lambda i: (0, 0)),
            pl.BlockSpec((H, H), lambda i: (0, 0)),
            pl.BlockSpec((H, H), lambda i: (0, 0)),
        ],
        out_specs=[
            pl.BlockSpec((BN, H), lambda i: (i, 0)),
            pl.BlockSpec((BN, H), lambda i: (i, 0)),
        ],
        out_shape=[
            jax.ShapeDtypeStruct((NP, H), jnp.float32),
            jax.ShapeDtypeStruct((NP, H), jnp.float32),
        ],
    )(accp, g, dis, b, Wa, Wb)


def _mlp_body(ghb, eab, wcb, b0b, w1b, b1b, w2b, b2b, ob):
    gh = ghb[...]
    z = gh[:, :H] + gh[:, H:] + jnp.dot(
        eab[...], wcb[...], preferred_element_type=jnp.float32) + b0b[...]
    z = jnp.maximum(z, 0.0)
    y = jnp.maximum(
        jnp.dot(z, w1b[...], preferred_element_type=jnp.float32) + b1b[...], 0.0)
    ob[...] = jnp.dot(y, w2b[...], preferred_element_type=jnp.float32) + b2b[...]


def _tc_mlp(GH, ea, Wc, b0, W1, b1, W2, b2):
    return pl.pallas_call(
        _mlp_body,
        grid=(E // BE,),
        in_specs=[
            pl.BlockSpec((BE, 2 * H), lambda i: (i, 0)),
            pl.BlockSpec((BE, F_E), lambda i: (i, 0)),
            pl.BlockSpec((F_E, H), lambda i: (0, 0)),
            pl.BlockSpec((1, H), lambda i: (0, 0)),
            pl.BlockSpec((H, H // 2), lambda i: (0, 0)),
            pl.BlockSpec((1, H // 2), lambda i: (0, 0)),
            pl.BlockSpec((H // 2, C), lambda i: (0, 0)),
            pl.BlockSpec((1, C), lambda i: (0, 0)),
        ],
        out_specs=pl.BlockSpec((BE, C), lambda i: (i, 0)),
        out_shape=jax.ShapeDtypeStruct((E, C), jnp.float32),
    )(GH, ea, Wc, b0, W1, b1, W2, b2)


def kernel(x, edge_index, edge_attr, W_enc, b_enc, W_c0, b_c0, W_c1, b_c1,
           W_c2, b_c2, W_p0, b_p0, W_p1, b_p1, W_p2, b_p2):
    src = edge_index[0]
    dst = edge_index[1]
    pad_e = IDXROWS * CHUNK - E
    srcp = jnp.concatenate([src, jnp.full((pad_e,), N, jnp.int32)]).reshape(-1, CHUNK)
    dstp = jnp.concatenate([dst, jnp.full((pad_e,), N, jnp.int32)]).reshape(-1, CHUNK)
    xp = jnp.pad(x, ((0, NP - N), (0, 0)))

    h0 = _tc_encoder(xp, W_enc, b_enc)
    degp = _sc_degree(dstp).reshape(NW, NP, 1)
    dis, g = _tc_layer0(degp, h0, W_c0)

    accp = _sc_conv(g, srcp, dstp)
    g = _tc_layer(accp, g, dis, b_c0.reshape(1, H), W_c1)
    accp = _sc_conv(g, srcp, dstp)
    g = _tc_layer(accp, g, dis, b_c1.reshape(1, H), W_c2)
    accp = _sc_conv(g, srcp, dstp)
    A, B = _tc_head(accp, g, dis, b_c2.reshape(1, H), W_p0[:H], W_p0[H:2 * H])

    GH = _sc_edge_gather(A, B, srcp, dstp)
    return _tc_mlp(GH, edge_attr, W_p0[2 * H:], b_p0.reshape(1, H),
                   W_p1, b_p1.reshape(1, H // 2), W_p2, b_p2.reshape(1, C))
